# Initial kernel scaffold; baseline (speedup 1.0000x reference)
import functools

import jax
import jax.numpy as jnp
from jax import lax
from jax.experimental import pallas as pl
from jax.experimental.pallas import tpu as pltpu
from jax.experimental.pallas import tpu_sc as plsc

N = 10000
E = 320000
IN_DIM = 128
HIDDEN = 32

NC = 2
NS = 16
NW = NC * NS
CHUNK = 128
K = -(-E // (NW * CHUNK))
E_PAD = NW * K * CHUNK
N_PAD = 10016
ROWS_W = N_PAD // NS
DEG_W = 16

_mesh = plsc.VectorSubcoreMesh(
    core_axis_name="c", subcore_axis_name="s", num_cores=NC, num_subcores=NS
)


@functools.partial(
    pl.kernel,
    out_type=jax.ShapeDtypeStruct((NC, N_PAD, DEG_W), jnp.float32),
    mesh=_mesh,
    scratch_types=[
        pltpu.VMEM((K, CHUNK), jnp.int32),
        pltpu.VMEM((CHUNK, DEG_W), jnp.float32),
        pltpu.VMEM_SHARED((N_PAD, DEG_W), jnp.float32),
    ],
)
def _deg_kernel(dst_hbm, ones_hbm, zeros_hbm, out_hbm, dst_v, ones_v, deg_sh):
    c = lax.axis_index("c")
    s = lax.axis_index("s")
    wid = c * NS + s
    pltpu.sync_copy(zeros_hbm, deg_sh.at[pl.ds(s * ROWS_W, ROWS_W)])
    pltpu.sync_copy(ones_hbm, ones_v)
    pltpu.sync_copy(dst_hbm.at[wid], dst_v)
    plsc.subcore_barrier()

    @pl.loop(0, K)
    def _(j):
        pltpu.sync_copy(ones_v, deg_sh.at[dst_v.at[j]], add=True)

    plsc.subcore_barrier()
    pltpu.sync_copy(
        deg_sh.at[pl.ds(s * ROWS_W, ROWS_W)],
        out_hbm.at[c, pl.ds(s * ROWS_W, ROWS_W)],
    )


@functools.partial(
    pl.kernel,
    out_type=jax.ShapeDtypeStruct((NC, N_PAD, HIDDEN), jnp.float32),
    mesh=_mesh,
    scratch_types=[
        pltpu.VMEM((K, CHUNK), jnp.int32),
        pltpu.VMEM((K, CHUNK), jnp.int32),
        pltpu.VMEM((CHUNK, HIDDEN), jnp.float32),
        pltpu.SemaphoreType.DMA,
        pltpu.VMEM_SHARED((N_PAD, HIDDEN), jnp.float32),
    ],
)
def _agg_kernel(g_hbm, src_hbm, dst_hbm, zeros_hbm, out_hbm,
                src_v, dst_v, rows_v, sem, acc_sh):
    c = lax.axis_index("c")
    s = lax.axis_index("s")
    wid = c * NS + s
    pltpu.sync_copy(zeros_hbm, acc_sh.at[pl.ds(s * ROWS_W, ROWS_W)])
    pltpu.sync_copy(src_hbm.at[wid], src_v)
    pltpu.sync_copy(dst_hbm.at[wid], dst_v)
    plsc.subcore_barrier()

    @pl.loop(0, K)
    def _(j):
        pltpu.async_copy(g_hbm.at[src_v.at[j]], rows_v, sem).wait()
        pltpu.sync_copy(rows_v, acc_sh.at[dst_v.at[j]], add=True)

    plsc.subcore_barrier()
    pltpu.sync_copy(
        acc_sh.at[pl.ds(s * ROWS_W, ROWS_W)],
        out_hbm.at[c, pl.ds(s * ROWS_W, ROWS_W)],
    )


_R = 2000


def _tc1_body(d0, d1, x, w1, dinv_out, g1_out):
    deg = d0[...][:, 0:1] + d1[...][:, 0:1] + 1.0
    dinv = lax.rsqrt(deg)
    h = jnp.dot(x[...], w1[...], preferred_element_type=jnp.float32)
    dinv_out[...] = dinv
    g1_out[...] = h * dinv


def _tc2_body(s0, s1, g1, dinv, b1, w2, g2_out):
    t = (s0[...] + s1[...] + g1[...]) * dinv[...] + b1[...]
    h = jnp.maximum(t, 0.0)
    g2_out[...] = jnp.dot(h, w2[...], preferred_element_type=jnp.float32) * dinv[...]


def _tc3_body(s0, s1, g2, dinv, b2, wl, bl, out):
    t = (s0[...] + s1[...] + g2[...]) * dinv[...] + b2[...]
    h = jnp.maximum(t, 0.0)
    out[...] = jnp.dot(h, wl[...], preferred_element_type=jnp.float32) + bl[...]


def _row_spec(w):
    return pl.BlockSpec((_R, w), lambda i: (i, 0))


def _full_spec(shape):
    return pl.BlockSpec(shape, lambda i: (0,) * len(shape))


_tc1 = pl.pallas_call(
    _tc1_body,
    grid=(N // _R,),
    in_specs=[
        _row_spec(DEG_W),
        _row_spec(DEG_W),
        _row_spec(IN_DIM),
        _full_spec((IN_DIM, HIDDEN)),
    ],
    out_specs=[_row_spec(1), _row_spec(HIDDEN)],
    out_shape=[
        jax.ShapeDtypeStruct((N, 1), jnp.float32),
        jax.ShapeDtypeStruct((N, HIDDEN), jnp.float32),
    ],
)

_tc2 = pl.pallas_call(
    _tc2_body,
    grid=(N // _R,),
    in_specs=[
        _row_spec(HIDDEN),
        _row_spec(HIDDEN),
        _row_spec(HIDDEN),
        _row_spec(1),
        _full_spec((1, HIDDEN)),
        _full_spec((HIDDEN, HIDDEN)),
    ],
    out_specs=_row_spec(HIDDEN),
    out_shape=jax.ShapeDtypeStruct((N, HIDDEN), jnp.float32),
)

_tc3 = pl.pallas_call(
    _tc3_body,
    grid=(N // _R,),
    in_specs=[
        _row_spec(HIDDEN),
        _row_spec(HIDDEN),
        _row_spec(HIDDEN),
        _row_spec(1),
        _full_spec((1, HIDDEN)),
        _full_spec((HIDDEN, 1)),
        _full_spec((1, 1)),
    ],
    out_specs=_row_spec(1),
    out_shape=jax.ShapeDtypeStruct((N, 1), jnp.float32),
)


def kernel(x, edge_index, W1, b1, W2, b2, Wl, bl):
    pad = E_PAD - E
    src3 = jnp.concatenate(
        [edge_index[0], jnp.zeros((pad,), jnp.int32)]).reshape(NW, K, CHUNK)
    dst3 = jnp.concatenate(
        [edge_index[1], jnp.full((pad,), N, jnp.int32)]).reshape(NW, K, CHUNK)
    ones_rows = jnp.ones((CHUNK, DEG_W), jnp.float32)
    zeros_deg = jnp.zeros((ROWS_W, DEG_W), jnp.float32)
    zeros_agg = jnp.zeros((ROWS_W, HIDDEN), jnp.float32)

    deg_parts = _deg_kernel(dst3, ones_rows, zeros_deg)
    dinv, g1 = _tc1(deg_parts[0, :N], deg_parts[1, :N], x, W1)

    s1 = _agg_kernel(g1, src3, dst3, zeros_agg)
    g2 = _tc2(s1[0, :N], s1[1, :N], g1, dinv, b1.reshape(1, HIDDEN), W2)

    s2 = _agg_kernel(g2, src3, dst3, zeros_agg)
    out = _tc3(s2[0, :N], s2[1, :N], g2, dinv, b2.reshape(1, HIDDEN),
               Wl, bl.reshape(1, 1))
    return out[:, 0]


# speedup vs baseline: 23.1616x; 23.1616x over previous
import functools

import jax
import jax.numpy as jnp
from jax import lax
from jax.experimental import pallas as pl
from jax.experimental.pallas import tpu as pltpu
from jax.experimental.pallas import tpu_sc as plsc

N = 10000
E = 320000
IN_DIM = 128
HIDDEN = 32

NC = 2
NS = 16
NW = NC * NS
CHUNK = 128
K = -(-E // (NW * CHUNK))
E_PAD = NW * K * CHUNK
N_PAD = 10112
ROWS_W = N_PAD // NS
DEG_W = 16

_mesh = plsc.VectorSubcoreMesh(
    core_axis_name="c", subcore_axis_name="s", num_cores=NC, num_subcores=NS
)


@functools.partial(
    pl.kernel,
    out_type=jax.ShapeDtypeStruct((NC, N_PAD, DEG_W), jnp.float32),
    mesh=_mesh,
    scratch_types=[
        pltpu.VMEM((K, CHUNK), jnp.int32),
        pltpu.VMEM((CHUNK, DEG_W), jnp.float32),
        pltpu.VMEM_SHARED((N_PAD, DEG_W), jnp.float32),
    ],
    compiler_params=pltpu.CompilerParams(use_tc_tiling_on_sc=False),
)
def _deg_kernel(dst_hbm, ones_hbm, zeros_hbm, out_hbm, dst_v, ones_v, deg_sh):
    c = lax.axis_index("c")
    s = lax.axis_index("s")
    wid = c * NS + s
    pltpu.sync_copy(zeros_hbm, deg_sh.at[pl.ds(s * ROWS_W, ROWS_W)])
    pltpu.sync_copy(ones_hbm, ones_v)
    pltpu.sync_copy(dst_hbm.at[wid], dst_v)
    plsc.subcore_barrier()

    @pl.loop(0, K)
    def _(j):
        pltpu.sync_copy(ones_v, deg_sh.at[dst_v.at[j]], add=True)

    plsc.subcore_barrier()
    pltpu.sync_copy(
        deg_sh.at[pl.ds(s * ROWS_W, ROWS_W)],
        out_hbm.at[c, pl.ds(s * ROWS_W, ROWS_W)],
    )


@functools.partial(
    pl.kernel,
    out_type=jax.ShapeDtypeStruct((NC, N_PAD, HIDDEN), jnp.float32),
    mesh=_mesh,
    scratch_types=[
        pltpu.VMEM((K, CHUNK), jnp.int32),
        pltpu.VMEM((K, CHUNK), jnp.int32),
        pltpu.VMEM((CHUNK, HIDDEN), jnp.float32),
        pltpu.SemaphoreType.DMA,
        pltpu.VMEM_SHARED((N_PAD, HIDDEN), jnp.float32),
    ],
    compiler_params=pltpu.CompilerParams(use_tc_tiling_on_sc=False),
)
def _agg_kernel(g_hbm, src_hbm, dst_hbm, zeros_hbm, out_hbm,
                src_v, dst_v, rows_v, sem, acc_sh):
    c = lax.axis_index("c")
    s = lax.axis_index("s")
    wid = c * NS + s
    pltpu.sync_copy(zeros_hbm, acc_sh.at[pl.ds(s * ROWS_W, ROWS_W)])
    pltpu.sync_copy(src_hbm.at[wid], src_v)
    pltpu.sync_copy(dst_hbm.at[wid], dst_v)
    plsc.subcore_barrier()

    @pl.loop(0, K)
    def _(j):
        pltpu.async_copy(g_hbm.at[src_v.at[j]], rows_v, sem).wait()
        pltpu.sync_copy(rows_v, acc_sh.at[dst_v.at[j]], add=True)

    plsc.subcore_barrier()
    pltpu.sync_copy(
        acc_sh.at[pl.ds(s * ROWS_W, ROWS_W)],
        out_hbm.at[c, pl.ds(s * ROWS_W, ROWS_W)],
    )


_R = 2000


def _tc1_body(d0, d1, x, w1, dinv_out, g1_out):
    deg = d0[...][:, 0:1] + d1[...][:, 0:1] + 1.0
    dinv = lax.rsqrt(deg)
    h = jnp.dot(x[...], w1[...], preferred_element_type=jnp.float32)
    dinv_out[...] = dinv
    g1_out[...] = h * dinv


def _tc2_body(s0, s1, g1, dinv, b1, w2, g2_out):
    t = (s0[...] + s1[...] + g1[...]) * dinv[...] + b1[...]
    h = jnp.maximum(t, 0.0)
    g2_out[...] = jnp.dot(h, w2[...], preferred_element_type=jnp.float32) * dinv[...]


def _tc3_body(s0, s1, g2, dinv, b2, wl, bl, out):
    t = (s0[...] + s1[...] + g2[...]) * dinv[...] + b2[...]
    h = jnp.maximum(t, 0.0)
    out[...] = jnp.dot(h, wl[...], preferred_element_type=jnp.float32) + bl[...]


def _row_spec(w):
    return pl.BlockSpec((_R, w), lambda i: (i, 0))


def _full_spec(shape):
    return pl.BlockSpec(shape, lambda i: (0,) * len(shape))


_tc1 = pl.pallas_call(
    _tc1_body,
    grid=(N // _R,),
    in_specs=[
        _row_spec(DEG_W),
        _row_spec(DEG_W),
        _row_spec(IN_DIM),
        _full_spec((IN_DIM, HIDDEN)),
    ],
    out_specs=[_row_spec(1), _row_spec(HIDDEN)],
    out_shape=[
        jax.ShapeDtypeStruct((N, 1), jnp.float32),
        jax.ShapeDtypeStruct((N, HIDDEN), jnp.float32),
    ],
)

_tc2 = pl.pallas_call(
    _tc2_body,
    grid=(N // _R,),
    in_specs=[
        _row_spec(HIDDEN),
        _row_spec(HIDDEN),
        _row_spec(HIDDEN),
        _row_spec(1),
        _full_spec((1, HIDDEN)),
        _full_spec((HIDDEN, HIDDEN)),
    ],
    out_specs=_row_spec(HIDDEN),
    out_shape=jax.ShapeDtypeStruct((N, HIDDEN), jnp.float32),
)

_tc3 = pl.pallas_call(
    _tc3_body,
    grid=(N // _R,),
    in_specs=[
        _row_spec(HIDDEN),
        _row_spec(HIDDEN),
        _row_spec(HIDDEN),
        _row_spec(1),
        _full_spec((1, HIDDEN)),
        _full_spec((HIDDEN, 1)),
        _full_spec((1, 1)),
    ],
    out_specs=_row_spec(1),
    out_shape=jax.ShapeDtypeStruct((N, 1), jnp.float32),
)


def kernel(x, edge_index, W1, b1, W2, b2, Wl, bl):
    pad = E_PAD - E
    src3 = jnp.concatenate(
        [edge_index[0], jnp.zeros((pad,), jnp.int32)]).reshape(NW, K, CHUNK)
    dst3 = jnp.concatenate(
        [edge_index[1], jnp.full((pad,), N, jnp.int32)]).reshape(NW, K, CHUNK)
    ones_rows = jnp.ones((CHUNK, DEG_W), jnp.float32)
    zeros_deg = jnp.zeros((ROWS_W, DEG_W), jnp.float32)
    zeros_agg = jnp.zeros((ROWS_W, HIDDEN), jnp.float32)

    deg_parts = _deg_kernel(dst3, ones_rows, zeros_deg)
    dinv, g1 = _tc1(deg_parts[0, :N], deg_parts[1, :N], x, W1)

    s1 = _agg_kernel(g1, src3, dst3, zeros_agg)
    g2 = _tc2(s1[0, :N], s1[1, :N], g1, dinv, b1.reshape(1, HIDDEN), W2)

    s2 = _agg_kernel(g2, src3, dst3, zeros_agg)
    out = _tc3(s2[0, :N], s2[1, :N], g2, dinv, b2.reshape(1, HIDDEN),
               Wl, bl.reshape(1, 1))
    return out[:, 0]
